# Initial kernel scaffold; baseline (speedup 1.0000x reference)
import jax
import jax.numpy as jnp
from jax.experimental import pallas as pl


def kernel(inp, edge_indices, edge_values):
    raise NotImplementedError("write your pallas kernel here")


# speedup vs baseline: 3.3043x; 3.3043x over previous
import functools

import jax
import jax.numpy as jnp
from jax import lax
from jax.experimental import pallas as pl
from jax.experimental.pallas import tpu as pltpu
from jax.experimental.pallas import tpu_sc as plsc

N_H = 16384
B = 64
N_IN = 512
N_OUT = 256
E = 268435
NT = 16
C = 128
CHUNKS = 136
E_PAD = NT * CHUNKS * C
U = 8
ROWS_PER_TILE = N_H // NT
WB = 128
LANES = 16


def _splat_i32(x):
    return jnp.full((LANES,), x, jnp.int32)


def _body(hs0, srcs, dsts, vals, zer, out, hs_a, hs_b,
          acc, src_t, dst_t, val_t, rows, sem):
    tid = lax.axis_index("s")

    pltpu.sync_copy(srcs.at[pl.ds(tid * CHUNKS, CHUNKS)], src_t)
    pltpu.sync_copy(dsts.at[pl.ds(tid * CHUNKS, CHUNKS)], dst_t)
    pltpu.sync_copy(vals.at[pl.ds(tid * CHUNKS * C, CHUNKS * C)], val_t)

    ins = [hs0, hs_a, hs_b]
    outs = [hs_a, hs_b, None]
    for step in range(3):
        hs_in = ins[step]

        pltpu.sync_copy(zer, acc.at[pl.ds(tid * ROWS_PER_TILE, ROWS_PER_TILE)])
        plsc.subcore_barrier()

        def chunk_body(c, carry):
            pltpu.async_copy(hs_in.at[src_t.at[c]], rows, sem).wait()

            def scale_body(i, carry2):
                for u in range(U):
                    e = i * U + u
                    vs = plsc.load_gather(val_t, [_splat_i32(c * C + e)])
                    for j in range(B // LANES):
                        sl = (e, pl.ds(j * LANES, LANES))
                        rows[sl] = rows[sl] * vs
                return carry2

            lax.fori_loop(0, C // U, scale_body, 0)
            pltpu.sync_copy(rows, acc.at[dst_t.at[c]], add=True)
            return carry

        lax.fori_loop(0, CHUNKS, chunk_body, 0)
        plsc.subcore_barrier()

        if step < 2:
            hs_out = outs[step]
            base = tid * ROWS_PER_TILE

            def wb_body(k, carry):
                rbase = base + k * WB
                pltpu.sync_copy(acc.at[pl.ds(rbase, WB)], rows)

                def relu_body(r4, carry2):
                    for u in range(4):
                        r = r4 * 4 + u
                        keep = _splat_i32(rbase + r) >= (N_H - N_OUT)
                        for j in range(B // LANES):
                            sl = (r, pl.ds(j * LANES, LANES))
                            x = rows[sl]
                            rows[sl] = jnp.where(keep, x, jnp.maximum(x, 0.0))
                    return carry2

                lax.fori_loop(0, WB // 4, relu_body, 0)
                pltpu.sync_copy(rows, hs_out.at[pl.ds(rbase, WB)])
                return carry

            lax.fori_loop(0, ROWS_PER_TILE // WB, wb_body, 0)
        else:
            @pl.when(tid == NT - 1)
            def _():
                for k in range(N_OUT // WB):
                    rbase = N_H - N_OUT + k * WB
                    pltpu.sync_copy(acc.at[pl.ds(rbase, WB)], rows)

                    def sig_body(r4, carry2):
                        for u in range(4):
                            r = r4 * 4 + u
                            for j in range(B // LANES):
                                sl = (r, pl.ds(j * LANES, LANES))
                                x = rows[sl]
                                rows[sl] = 1.0 / (1.0 + jnp.exp(-x))
                        return carry2

                    lax.fori_loop(0, WB // 4, sig_body, 0)
                    pltpu.sync_copy(rows, out.at[pl.ds(k * WB, WB)])


_sun_sc = functools.partial(
    pl.kernel,
    out_type=(
        jax.ShapeDtypeStruct((N_OUT, B), jnp.float32),
        jax.ShapeDtypeStruct((N_H, B), jnp.float32),
        jax.ShapeDtypeStruct((N_H, B), jnp.float32),
    ),
    mesh=plsc.VectorSubcoreMesh(
        core_axis_name="c", subcore_axis_name="s", num_cores=1
    ),
    compiler_params=pltpu.CompilerParams(
        needs_layout_passes=False, use_tc_tiling_on_sc=False
    ),
    scratch_types=[
        pltpu.VMEM_SHARED((N_H, B), jnp.float32),
        pltpu.VMEM((CHUNKS, C), jnp.int32),
        pltpu.VMEM((CHUNKS, C), jnp.int32),
        pltpu.VMEM((CHUNKS * C,), jnp.float32),
        pltpu.VMEM((C, B), jnp.float32),
        pltpu.SemaphoreType.DMA,
    ],
)(_body)


@jax.jit
def kernel(inp, edge_indices, edge_values):
    src = edge_indices[0].astype(jnp.int32)
    dst = edge_indices[1].astype(jnp.int32)
    val = edge_values.astype(jnp.float32)
    pad = E_PAD - E
    src = jnp.pad(src, (0, pad)).reshape(NT * CHUNKS, C)
    dst = jnp.pad(dst, (0, pad)).reshape(NT * CHUNKS, C)
    val = jnp.pad(val, (0, pad))
    hs0 = jnp.zeros((N_H, B), jnp.float32).at[:N_IN].set(inp.T)
    zer = jnp.zeros((ROWS_PER_TILE, B), jnp.float32)
    out, _, _ = _sun_sc(hs0, src, dst, val, zer)
    return out.T


# speedup vs baseline: 5.6485x; 1.7094x over previous
import functools

import jax
import jax.numpy as jnp
import numpy as np
from jax import lax
from jax.experimental import pallas as pl
from jax.experimental.pallas import tpu as pltpu
from jax.experimental.pallas import tpu_sc as plsc

N_H = 16384
B = 64
HB = B // 2
N_IN = 512
N_OUT = 256
E = 268435
NT = 16
C = 128
CHUNKS = 136
E_PAD = NT * CHUNKS * C
U = 8
ROWS_PER_TILE = N_H // NT
WB = 128
LANES = 16
JV = HB // LANES


def _splat_i32(x):
    return jnp.full((LANES,), x, jnp.int32)


def _body(hs0, srcs, dsts, vals, zer, out, hs_a, hs_b,
          acc, src_t, dst_t, val_t, rows, sem):
    cid = lax.axis_index("c")
    tid = lax.axis_index("s")

    pltpu.sync_copy(srcs.at[pl.ds(tid * CHUNKS, CHUNKS)], src_t)
    pltpu.sync_copy(dsts.at[pl.ds(tid * CHUNKS, CHUNKS)], dst_t)
    pltpu.sync_copy(vals.at[pl.ds(tid * CHUNKS * C, CHUNKS * C)], val_t)

    ins = [hs0, hs_a, hs_b]
    outs = [hs_a, hs_b, None]
    for step in range(3):
        hs_in = ins[step].at[cid]

        pltpu.sync_copy(zer, acc.at[pl.ds(tid * ROWS_PER_TILE, ROWS_PER_TILE)])
        plsc.subcore_barrier()

        def process_chunk(c):
            pltpu.async_copy(hs_in.at[src_t.at[c]], rows, sem).wait()

            def scale_body(i, carry2):
                for u in range(U):
                    e = i * U + u
                    vs = plsc.load_gather(val_t, [_splat_i32(c * C + e)])
                    for j in range(JV):
                        sl = (e, pl.ds(j * LANES, LANES))
                        rows[sl] = rows[sl] * vs
                return carry2

            lax.fori_loop(0, C // U, scale_body, 0)
            pltpu.sync_copy(rows, acc.at[dst_t.at[c]], add=True)

        if step == 0:
            def chunk_body(c, carry):
                smin = jnp.min(src_t[c, pl.ds(0, LANES)])

                @pl.when(smin < N_IN)
                def _():
                    process_chunk(c)

                return carry
        else:
            def chunk_body(c, carry):
                process_chunk(c)
                return carry

        lax.fori_loop(0, CHUNKS, chunk_body, 0)
        plsc.subcore_barrier()

        if step < 2:
            hs_out = outs[step].at[cid]
            base = tid * ROWS_PER_TILE

            def wb_body(k, carry):
                rbase = base + k * WB
                pltpu.sync_copy(acc.at[pl.ds(rbase, WB)], rows)

                def relu_body(r4, carry2):
                    for u in range(4):
                        r = r4 * 4 + u
                        keep = _splat_i32(rbase + r) >= (N_H - N_OUT)
                        for j in range(JV):
                            sl = (r, pl.ds(j * LANES, LANES))
                            x = rows[sl]
                            rows[sl] = jnp.where(keep, x, jnp.maximum(x, 0.0))
                    return carry2

                lax.fori_loop(0, WB // 4, relu_body, 0)
                pltpu.sync_copy(rows, hs_out.at[pl.ds(rbase, WB)])
                return carry

            lax.fori_loop(0, ROWS_PER_TILE // WB, wb_body, 0)
        else:
            @pl.when(tid == NT - 1)
            def _():
                for k in range(N_OUT // WB):
                    rbase = N_H - N_OUT + k * WB
                    pltpu.sync_copy(acc.at[pl.ds(rbase, WB)], rows)

                    def sig_body(r4, carry2):
                        for u in range(4):
                            r = r4 * 4 + u
                            for j in range(JV):
                                sl = (r, pl.ds(j * LANES, LANES))
                                x = rows[sl]
                                rows[sl] = 1.0 / (1.0 + jnp.exp(-x))
                        return carry2

                    lax.fori_loop(0, WB // 4, sig_body, 0)
                    pltpu.sync_copy(rows, out.at[cid].at[pl.ds(k * WB, WB)])


_sun_sc = functools.partial(
    pl.kernel,
    out_type=(
        jax.ShapeDtypeStruct((2, N_OUT, HB), jnp.float32),
        jax.ShapeDtypeStruct((2, N_H, HB), jnp.float32),
        jax.ShapeDtypeStruct((2, N_H, HB), jnp.float32),
    ),
    mesh=plsc.VectorSubcoreMesh(core_axis_name="c", subcore_axis_name="s"),
    compiler_params=pltpu.CompilerParams(
        needs_layout_passes=False, use_tc_tiling_on_sc=False
    ),
    scratch_types=[
        pltpu.VMEM_SHARED((N_H, HB), jnp.float32),
        pltpu.VMEM((CHUNKS, C), jnp.int32),
        pltpu.VMEM((CHUNKS, C), jnp.int32),
        pltpu.VMEM((CHUNKS * C,), jnp.float32),
        pltpu.VMEM((C, HB), jnp.float32),
        pltpu.SemaphoreType.DMA,
    ],
)(_body)

_PERM = np.arange(NT * CHUNKS).reshape(CHUNKS, NT).T.reshape(-1)


@jax.jit
def kernel(inp, edge_indices, edge_values):
    src = edge_indices[0].astype(jnp.int32)
    dst = edge_indices[1].astype(jnp.int32)
    val = edge_values.astype(jnp.float32)
    pad = E_PAD - E
    src = jnp.pad(src, (0, pad)).reshape(NT * CHUNKS, C)[_PERM]
    dst = jnp.pad(dst, (0, pad)).reshape(NT * CHUNKS, C)[_PERM]
    val = jnp.pad(val, (0, pad)).reshape(NT * CHUNKS, C)[_PERM].reshape(-1)
    hs0 = jnp.zeros((N_H, B), jnp.float32).at[:N_IN].set(inp.T)
    hs0 = hs0.reshape(N_H, 2, HB).transpose(1, 0, 2)
    zer = jnp.zeros((ROWS_PER_TILE, HB), jnp.float32)
    out, _, _ = _sun_sc(hs0, src, dst, val, zer)
    return jnp.concatenate([out[0], out[1]], axis=1).T


# speedup vs baseline: 10.1854x; 1.8032x over previous
import functools

import jax
import jax.numpy as jnp
import numpy as np
from jax import lax
from jax.experimental import pallas as pl
from jax.experimental.pallas import tpu as pltpu
from jax.experimental.pallas import tpu_sc as plsc

N_H = 16384
B = 64
HB = B // 2
N_IN = 512
N_OUT = 256
E = 268435
NT = 16
C = 128
CHUNKS = 136
E_PAD = NT * CHUNKS * C
NBUF = 4
ROWS_PER_TILE = N_H // NT
WB = 128
LANES = 16
JV = HB // LANES


def _splat_i32(x):
    return jnp.full((LANES,), x, jnp.int32)


def _body(hs0, srcs, dsts, vals, zer, out, hs_a, hs_b, acc,
          src_t, dst_t, val_t, r0, r1, r2, r3,
          g0, g1, g2, g3, s0, s1, s2, s3):
    rows = [r0, r1, r2, r3]
    semg = [g0, g1, g2, g3]
    sems = [s0, s1, s2, s3]
    cid = lax.axis_index("c")
    tid = lax.axis_index("s")

    pltpu.sync_copy(srcs.at[pl.ds(tid * CHUNKS, CHUNKS)], src_t)
    pltpu.sync_copy(dsts.at[pl.ds(tid * CHUNKS, CHUNKS)], dst_t)
    pltpu.sync_copy(vals.at[pl.ds(tid * CHUNKS * C, CHUNKS * C)], val_t)

    def scale(b, c):
        def scale_body(i, carry2):
            vv = val_t[pl.ds(c * C + i * LANES, LANES)]
            for u in range(LANES):
                e = i * LANES + u
                vs = jnp.take_along_axis(
                    vv, jnp.full((LANES,), u, jnp.int32), axis=0)
                for j in range(JV):
                    sl = (e, pl.ds(j * LANES, LANES))
                    rows[b][sl] = rows[b][sl] * vs
            return carry2

        lax.fori_loop(0, C // LANES, scale_body, 0)

    ins = [hs0, hs_a, hs_b]
    outs = [hs_a, hs_b, None]
    for step in range(3):
        hs_in = ins[step].at[cid]

        pltpu.sync_copy(zer, acc.at[pl.ds(tid * ROWS_PER_TILE, ROWS_PER_TILE)])
        plsc.subcore_barrier()

        if step == 0:
            def chunk_body(c, carry):
                smin = jnp.min(src_t[c, pl.ds(0, LANES)])

                @pl.when(smin < N_IN)
                def _():
                    pltpu.async_copy(hs_in.at[src_t.at[c]], rows[0], semg[0]
                                     ).wait()
                    scale(0, c)
                    pltpu.sync_copy(rows[0], acc.at[dst_t.at[c]], add=True)

                return carry

            lax.fori_loop(0, CHUNKS, chunk_body, 0)
        else:
            def wait_scatter(b):
                pltpu.make_async_copy(
                    rows[b], acc.at[dst_t.at[0]], sems[b]).wait()

            def start_gather(b, c):
                pltpu.async_copy(hs_in.at[src_t.at[c]], rows[b], semg[b])

            def section(b, c):
                pltpu.make_async_copy(
                    hs_in.at[src_t.at[c]], rows[b], semg[b]).wait()
                scale(b, c)
                pltpu.async_copy(rows[b], acc.at[dst_t.at[c]], sems[b],
                                 add=True)

            for b in range(NBUF - 1):
                start_gather(b, jnp.int32(b))
            for b in range(NBUF):
                section(b, jnp.int32(b))
                bp = (b + NBUF - 1) % NBUF
                if b > 0:
                    wait_scatter(bp)
                start_gather(bp, jnp.int32(b + NBUF - 1))

            def chunk4_body(i, carry):
                for b in range(NBUF):
                    c = i * NBUF + b
                    section(b, c)
                    bp = (b + NBUF - 1) % NBUF
                    nxt = c + NBUF - 1

                    @pl.when(nxt < CHUNKS)
                    def _():
                        wait_scatter(bp)
                        start_gather(bp, nxt)

                return carry

            lax.fori_loop(1, CHUNKS // NBUF, chunk4_body, 0)
            for b in range(NBUF):
                wait_scatter(b)

        plsc.subcore_barrier()

        if step < 2:
            hs_out = outs[step].at[cid]
            base = tid * ROWS_PER_TILE

            def wb_body(k, carry):
                rbase = base + k * WB
                pltpu.sync_copy(acc.at[pl.ds(rbase, WB)], rows[0])

                def relu_body(r4, carry2):
                    for u in range(4):
                        r = r4 * 4 + u
                        keep = _splat_i32(rbase + r) >= (N_H - N_OUT)
                        for j in range(JV):
                            sl = (r, pl.ds(j * LANES, LANES))
                            x = rows[0][sl]
                            rows[0][sl] = jnp.where(keep, x,
                                                    jnp.maximum(x, 0.0))
                    return carry2

                lax.fori_loop(0, WB // 4, relu_body, 0)
                pltpu.sync_copy(rows[0], hs_out.at[pl.ds(rbase, WB)])
                return carry

            lax.fori_loop(0, ROWS_PER_TILE // WB, wb_body, 0)
        else:
            @pl.when(tid == NT - 1)
            def _():
                for k in range(N_OUT // WB):
                    rbase = N_H - N_OUT + k * WB
                    pltpu.sync_copy(acc.at[pl.ds(rbase, WB)], rows[0])

                    def sig_body(r4, carry2):
                        for u in range(4):
                            r = r4 * 4 + u
                            for j in range(JV):
                                sl = (r, pl.ds(j * LANES, LANES))
                                x = rows[0][sl]
                                rows[0][sl] = 1.0 / (1.0 + jnp.exp(-x))
                        return carry2

                    lax.fori_loop(0, WB // 4, sig_body, 0)
                    pltpu.sync_copy(rows[0], out.at[cid].at[pl.ds(k * WB, WB)])


_sun_sc = functools.partial(
    pl.kernel,
    out_type=(
        jax.ShapeDtypeStruct((2, N_OUT, HB), jnp.float32),
        jax.ShapeDtypeStruct((2, N_H, HB), jnp.float32),
        jax.ShapeDtypeStruct((2, N_H, HB), jnp.float32),
    ),
    mesh=plsc.VectorSubcoreMesh(core_axis_name="c", subcore_axis_name="s"),
    compiler_params=pltpu.CompilerParams(
        needs_layout_passes=False, use_tc_tiling_on_sc=False
    ),
    scratch_types=[
        pltpu.VMEM_SHARED((N_H, HB), jnp.float32),
        pltpu.VMEM((CHUNKS, C), jnp.int32),
        pltpu.VMEM((CHUNKS, C), jnp.int32),
        pltpu.VMEM((CHUNKS * C,), jnp.float32),
        pltpu.VMEM((C, HB), jnp.float32),
        pltpu.VMEM((C, HB), jnp.float32),
        pltpu.VMEM((C, HB), jnp.float32),
        pltpu.VMEM((C, HB), jnp.float32),
        pltpu.SemaphoreType.DMA,
        pltpu.SemaphoreType.DMA,
        pltpu.SemaphoreType.DMA,
        pltpu.SemaphoreType.DMA,
        pltpu.SemaphoreType.DMA,
        pltpu.SemaphoreType.DMA,
        pltpu.SemaphoreType.DMA,
        pltpu.SemaphoreType.DMA,
    ],
)(_body)

_PERM = np.arange(NT * CHUNKS).reshape(CHUNKS, NT).T.reshape(-1)


@jax.jit
def kernel(inp, edge_indices, edge_values):
    src = edge_indices[0].astype(jnp.int32)
    dst = edge_indices[1].astype(jnp.int32)
    val = edge_values.astype(jnp.float32)
    pad = E_PAD - E
    src = jnp.pad(src, (0, pad)).reshape(NT * CHUNKS, C)[_PERM]
    dst = jnp.pad(dst, (0, pad)).reshape(NT * CHUNKS, C)[_PERM]
    val = jnp.pad(val, (0, pad)).reshape(NT * CHUNKS, C)[_PERM].reshape(-1)
    hs0 = jnp.zeros((N_H, B), jnp.float32).at[:N_IN].set(inp.T)
    hs0 = hs0.reshape(N_H, 2, HB).transpose(1, 0, 2)
    zer = jnp.zeros((ROWS_PER_TILE, HB), jnp.float32)
    out, _, _ = _sun_sc(hs0, src, dst, val, zer)
    return jnp.concatenate([out[0], out[1]], axis=1).T


# speedup vs baseline: 13.6366x; 1.3388x over previous
import functools

import jax
import jax.numpy as jnp
import numpy as np
from jax import lax
from jax.experimental import pallas as pl
from jax.experimental.pallas import tpu as pltpu
from jax.experimental.pallas import tpu_sc as plsc

N_H = 16384
B = 64
HB = B // 2
N_IN = 512
N_OUT = 256
E = 268435
NT = 16
C = 128
CHUNKS = 136
E_PAD = NT * CHUNKS * C
NBUF = 4
CAP3 = 2176
ROWS_PER_TILE = N_H // NT
WB = 128
LANES = 16
JV = HB // LANES


def _splat_i32(x):
    return jnp.full((LANES,), x, jnp.int32)


def _body(hs0, srcs, dsts, vals, zer, out, hs_a, hs_b, acc,
          src_t, dst_t, val_t, c_src, c_dst, c_val, rows16,
          r0, r1, r2, r3, g0, g1, g2, g3, s0, s1, s2, s3):
    rows = [r0, r1, r2, r3]
    semg = [g0, g1, g2, g3]
    sems = [s0, s1, s2, s3]
    cid = lax.axis_index("c")
    tid = lax.axis_index("s")

    pltpu.sync_copy(srcs.at[pl.ds(tid * CHUNKS, CHUNKS)], src_t)
    pltpu.sync_copy(dsts.at[pl.ds(tid * CHUNKS, CHUNKS)], dst_t)
    pltpu.sync_copy(vals.at[pl.ds(tid * CHUNKS * C, CHUNKS * C)], val_t)

    zi = jnp.zeros((LANES,), jnp.int32)
    zf = jnp.zeros((LANES,), jnp.float32)

    def z_body(i, carry):
        c_src[pl.ds(i * LANES, LANES)] = zi
        c_dst[pl.ds(i * LANES, LANES)] = zi
        c_val[pl.ds(i * LANES, LANES)] = zf
        return carry

    lax.fori_loop(0, CAP3 // LANES, z_body, 0)

    def cp_body(g, cnt):
        c = g // (C // LANES)
        off = (g % (C // LANES)) * LANES
        d = dst_t[c, pl.ds(off, LANES)]
        m = d >= N_H - N_OUT
        npop = jnp.sum(m.astype(jnp.int32))
        ok = (cnt + npop) <= CAP3

        @pl.when(jnp.logical_and(ok, npop > 0))
        def _():
            s = src_t[c, pl.ds(off, LANES)]
            v = val_t[pl.ds(g * LANES, LANES)]
            plsc.store_compressed(c_dst.at[pl.ds(cnt, LANES)], d, mask=m)
            plsc.store_compressed(c_src.at[pl.ds(cnt, LANES)], s, mask=m)
            plsc.store_compressed(c_val.at[pl.ds(cnt, LANES)], v, mask=m)

        return jnp.where(ok, cnt + npop, jnp.int32(CAP3 + 1))

    n3 = lax.fori_loop(0, CHUNKS * C // LANES, cp_body, jnp.int32(0))

    def scale(b, c):
        def scale_body(i, carry2):
            vv = val_t[pl.ds(c * C + i * LANES, LANES)]
            for u in range(LANES):
                e = i * LANES + u
                vs = jnp.take_along_axis(
                    vv, jnp.full((LANES,), u, jnp.int32), axis=0)
                for j in range(JV):
                    sl = (e, pl.ds(j * LANES, LANES))
                    rows[b][sl] = rows[b][sl] * vs
            return carry2

        lax.fori_loop(0, C // LANES, scale_body, 0)

    ins = [hs0, hs_a, hs_b]
    outs = [hs_a, hs_b, None]
    for step in range(3):
        hs_in = ins[step].at[cid]

        pltpu.sync_copy(zer, acc.at[pl.ds(tid * ROWS_PER_TILE, ROWS_PER_TILE)])
        plsc.subcore_barrier()

        if step == 0:
            def chunk_body(c, carry):
                smin = jnp.min(src_t[c, pl.ds(0, LANES)])

                @pl.when(smin < N_IN)
                def _():
                    pltpu.async_copy(hs_in.at[src_t.at[c]], rows[0], semg[0]
                                     ).wait()
                    scale(0, c)
                    pltpu.sync_copy(rows[0], acc.at[dst_t.at[c]], add=True)

                return carry

            lax.fori_loop(0, CHUNKS, chunk_body, 0)
        elif step == 2:
            @pl.when(n3 <= CAP3)
            def _():
                ng = (n3 + LANES - 1) // LANES

                def g_body(g, carry):
                    svec = c_src[pl.ds(g * LANES, LANES)]
                    dvec = c_dst[pl.ds(g * LANES, LANES)]
                    vvec = c_val[pl.ds(g * LANES, LANES)]
                    pltpu.async_copy(hs_in.at[svec], rows16, semg[0]).wait()
                    for u in range(LANES):
                        vs = jnp.take_along_axis(
                            vvec, jnp.full((LANES,), u, jnp.int32), axis=0)
                        for j in range(JV):
                            sl = (u, pl.ds(j * LANES, LANES))
                            rows16[sl] = rows16[sl] * vs
                    pltpu.sync_copy(rows16, acc.at[dvec], add=True)
                    return carry

                lax.fori_loop(0, ng, g_body, 0)

            @pl.when(n3 > CAP3)
            def _():
                def full_body(c, carry):
                    pltpu.async_copy(hs_in.at[src_t.at[c]], rows[0], semg[0]
                                     ).wait()
                    scale(0, c)
                    pltpu.sync_copy(rows[0], acc.at[dst_t.at[c]], add=True)
                    return carry

                lax.fori_loop(0, CHUNKS, full_body, 0)
        else:
            def wait_scatter(b):
                pltpu.make_async_copy(
                    rows[b], acc.at[dst_t.at[0]], sems[b]).wait()

            def start_gather(b, c):
                pltpu.async_copy(hs_in.at[src_t.at[c]], rows[b], semg[b])

            def section(b, c):
                pltpu.make_async_copy(
                    hs_in.at[src_t.at[c]], rows[b], semg[b]).wait()
                scale(b, c)
                pltpu.async_copy(rows[b], acc.at[dst_t.at[c]], sems[b],
                                 add=True)

            for b in range(NBUF - 1):
                start_gather(b, jnp.int32(b))
            for b in range(NBUF):
                section(b, jnp.int32(b))
                bp = (b + NBUF - 1) % NBUF
                if b > 0:
                    wait_scatter(bp)
                start_gather(bp, jnp.int32(b + NBUF - 1))

            def chunk4_body(i, carry):
                for b in range(NBUF):
                    c = i * NBUF + b
                    section(b, c)
                    bp = (b + NBUF - 1) % NBUF
                    nxt = c + NBUF - 1

                    @pl.when(nxt < CHUNKS)
                    def _():
                        wait_scatter(bp)
                        start_gather(bp, nxt)

                return carry

            lax.fori_loop(1, CHUNKS // NBUF, chunk4_body, 0)
            for b in range(NBUF):
                wait_scatter(b)

        plsc.subcore_barrier()

        if step < 2:
            hs_out = outs[step].at[cid]
            base = tid * ROWS_PER_TILE

            def wb_body(k, carry):
                rbase = base + k * WB
                pltpu.sync_copy(acc.at[pl.ds(rbase, WB)], rows[0])

                def relu_body(r4, carry2):
                    for u in range(4):
                        r = r4 * 4 + u
                        keep = _splat_i32(rbase + r) >= (N_H - N_OUT)
                        for j in range(JV):
                            sl = (r, pl.ds(j * LANES, LANES))
                            x = rows[0][sl]
                            rows[0][sl] = jnp.where(keep, x,
                                                    jnp.maximum(x, 0.0))
                    return carry2

                lax.fori_loop(0, WB // 4, relu_body, 0)
                pltpu.sync_copy(rows[0], hs_out.at[pl.ds(rbase, WB)])
                return carry

            lax.fori_loop(0, ROWS_PER_TILE // WB, wb_body, 0)
        else:
            @pl.when(tid == NT - 1)
            def _():
                for k in range(N_OUT // WB):
                    rbase = N_H - N_OUT + k * WB
                    pltpu.sync_copy(acc.at[pl.ds(rbase, WB)], rows[0])

                    def sig_body(r4, carry2):
                        for u in range(4):
                            r = r4 * 4 + u
                            for j in range(JV):
                                sl = (r, pl.ds(j * LANES, LANES))
                                x = rows[0][sl]
                                rows[0][sl] = 1.0 / (1.0 + jnp.exp(-x))
                        return carry2

                    lax.fori_loop(0, WB // 4, sig_body, 0)
                    pltpu.sync_copy(rows[0], out.at[cid].at[pl.ds(k * WB, WB)])


_sun_sc = functools.partial(
    pl.kernel,
    out_type=(
        jax.ShapeDtypeStruct((2, N_OUT, HB), jnp.float32),
        jax.ShapeDtypeStruct((2, N_H, HB), jnp.float32),
        jax.ShapeDtypeStruct((2, N_H, HB), jnp.float32),
    ),
    mesh=plsc.VectorSubcoreMesh(core_axis_name="c", subcore_axis_name="s"),
    compiler_params=pltpu.CompilerParams(
        needs_layout_passes=False, use_tc_tiling_on_sc=False
    ),
    scratch_types=[
        pltpu.VMEM_SHARED((N_H, HB), jnp.float32),
        pltpu.VMEM((CHUNKS, C), jnp.int32),
        pltpu.VMEM((CHUNKS, C), jnp.int32),
        pltpu.VMEM((CHUNKS * C,), jnp.float32),
        pltpu.VMEM((CAP3,), jnp.int32),
        pltpu.VMEM((CAP3,), jnp.int32),
        pltpu.VMEM((CAP3,), jnp.float32),
        pltpu.VMEM((LANES, HB), jnp.float32),
        pltpu.VMEM((C, HB), jnp.float32),
        pltpu.VMEM((C, HB), jnp.float32),
        pltpu.VMEM((C, HB), jnp.float32),
        pltpu.VMEM((C, HB), jnp.float32),
        pltpu.SemaphoreType.DMA,
        pltpu.SemaphoreType.DMA,
        pltpu.SemaphoreType.DMA,
        pltpu.SemaphoreType.DMA,
        pltpu.SemaphoreType.DMA,
        pltpu.SemaphoreType.DMA,
        pltpu.SemaphoreType.DMA,
        pltpu.SemaphoreType.DMA,
    ],
)(_body)

_PERM = np.arange(NT * CHUNKS).reshape(CHUNKS, NT).T.reshape(-1)


@jax.jit
def kernel(inp, edge_indices, edge_values):
    src = edge_indices[0].astype(jnp.int32)
    dst = edge_indices[1].astype(jnp.int32)
    val = edge_values.astype(jnp.float32)
    pad = E_PAD - E
    src = jnp.pad(src, (0, pad)).reshape(NT * CHUNKS, C)[_PERM]
    dst = jnp.pad(dst, (0, pad)).reshape(NT * CHUNKS, C)[_PERM]
    val = jnp.pad(val, (0, pad)).reshape(NT * CHUNKS, C)[_PERM].reshape(-1)
    hs0 = jnp.zeros((N_H, B), jnp.float32).at[:N_IN].set(inp.T)
    hs0 = hs0.reshape(N_H, 2, HB).transpose(1, 0, 2)
    zer = jnp.zeros((ROWS_PER_TILE, HB), jnp.float32)
    out, _, _ = _sun_sc(hs0, src, dst, val, zer)
    return jnp.concatenate([out[0], out[1]], axis=1).T
